# baseline (speedup 1.0000x reference)
import functools
import jax
import jax.numpy as jnp
from jax import lax
from jax.experimental import pallas as pl
from jax.experimental.pallas import tpu as pltpu
from jax.experimental.pallas import tpu_sc as plsc

N = 10000
E = 320000
D = 128
H = 128
C = 40
CP = 64

NC, NS = 2, 16
CH = 128
NCHT = 160
QC = 20
E_PAD = NS * NCHT * CH
N_ACC = 10112
ROWS_PT = N_ACC // NS
TROWS_PT = N // NS


@functools.lru_cache(maxsize=None)
def _seg_sum_kernel(FH, nbuf):
  assert QC % nbuf == 0 and NCHT % QC == 0
  mesh = plsc.VectorSubcoreMesh(
      core_axis_name="c", subcore_axis_name="s", num_cores=NC, num_subcores=NS)

  @functools.partial(
      pl.kernel,
      out_type=jax.ShapeDtypeStruct((NC, N_ACC, FH), jnp.float32),
      mesh=mesh,
      scratch_types=(
          [
              pltpu.VMEM((QC, CH), jnp.int32),
              pltpu.VMEM((QC, CH), jnp.int32),
          ]
          + [pltpu.VMEM((CH, FH), jnp.float32) for _ in range(nbuf)]
          + [
              pltpu.VMEM_SHARED((N, FH), jnp.float32),
              pltpu.VMEM_SHARED((N_ACC, FH), jnp.float32),
          ]
          + [pltpu.SemaphoreType.DMA for _ in range(nbuf)]
      ),
      compiler_params=pltpu.CompilerParams(use_tc_tiling_on_sc=False),
  )
  def seg_sum(h2_hbm, src_hbm, dst_hbm, out_hbm, src_v, dst_v, *rest):
    bufs = rest[:nbuf]
    table = rest[nbuf]
    acc = rest[nbuf + 1]
    sems = rest[nbuf + 2:]
    c = lax.axis_index("c")
    s = lax.axis_index("s")

    zv = jnp.zeros((16,), jnp.float32)

    with jax.named_scope("stage"):
      def zrow(i, carry):
        for k in range(FH // 16):
          bufs[0][i, pl.ds(k * 16, 16)] = zv
        return carry

      lax.fori_loop(0, CH, zrow, 0)

      base = s * ROWS_PT
      for r in range(ROWS_PT // CH):
        pltpu.sync_copy(bufs[0], acc.at[pl.ds(base + r * CH, CH)])
      rem = ROWS_PT % CH
      if rem:
        pltpu.sync_copy(bufs[0].at[pl.ds(0, rem)],
                        acc.at[pl.ds(base + (ROWS_PT // CH) * CH, rem)])

      tbase = s * TROWS_PT
      pltpu.sync_copy(h2_hbm.at[pl.ds(tbase, TROWS_PT), pl.ds(c * FH, FH)],
                      table.at[pl.ds(tbase, TROWS_PT)])

      plsc.subcore_barrier()

    def wait_gather(k):
      pltpu.make_async_copy(table.at[pl.ds(0, CH)], bufs[k], sems[k]).wait()

    with jax.named_scope("edges"):
      for g in range(NCHT // QC):
        chunk0 = s * NCHT + g * QC
        pltpu.sync_copy(src_hbm.at[pl.ds(chunk0, QC)], src_v)
        pltpu.sync_copy(dst_hbm.at[pl.ds(chunk0, QC)], dst_v)

        for k in range(nbuf):
          pltpu.async_copy(table.at[src_v.at[k]], bufs[k], sems[k])

        def body(q, carry):
          j = q * nbuf
          for k in range(nbuf):
            wait_gather(k)
            pltpu.sync_copy(bufs[k], acc.at[dst_v.at[j + k]], add=True)
            pltpu.async_copy(table.at[src_v.at[j + k + nbuf]], bufs[k], sems[k])
          return carry

        lax.fori_loop(0, QC // nbuf - 1, body, 0)

        j = QC - nbuf
        for k in range(nbuf):
          wait_gather(k)
          pltpu.sync_copy(bufs[k], acc.at[dst_v.at[j + k]], add=True)

      plsc.subcore_barrier()

    with jax.named_scope("copyout"):
      pltpu.sync_copy(acc.at[pl.ds(base, ROWS_PT)],
                      out_hbm.at[c, pl.ds(base, ROWS_PT)])

  return seg_sum


def _mlp_body(x_ref, a0_ref, a1_ref, w1_ref, b1_ref, w2_ref, m_ref):
  agg = jnp.concatenate([a0_ref[...], a1_ref[...]], axis=1)
  t = x_ref[...] + agg
  h = jnp.dot(t, w1_ref[...], preferred_element_type=jnp.float32) + b1_ref[...]
  h = jnp.maximum(h, 0.0)
  m_ref[...] = jnp.dot(h, w2_ref[...], preferred_element_type=jnp.float32)


def _mlp(x, a0, a1, w1, b1, w2p):
  blk = 1000
  grid = (N // blk,)
  hd = D // 2
  return pl.pallas_call(
      _mlp_body,
      grid=grid,
      in_specs=[
          pl.BlockSpec((blk, D), lambda i: (i, 0)),
          pl.BlockSpec((blk, hd), lambda i: (i, 0)),
          pl.BlockSpec((blk, hd), lambda i: (i, 0)),
          pl.BlockSpec((D, H), lambda i: (0, 0)),
          pl.BlockSpec((1, H), lambda i: (0, 0)),
          pl.BlockSpec((H, CP), lambda i: (0, 0)),
      ],
      out_specs=pl.BlockSpec((blk, CP), lambda i: (i, 0)),
      out_shape=jax.ShapeDtypeStruct((N, CP), jnp.float32),
  )(x, a0, a1, w1, b1, w2p)


def _final_body(m_ref, a0_ref, a1_ref, b2_ref, o_ref):
  agg = jnp.concatenate([a0_ref[...], a1_ref[...][:, :C - CP // 2]], axis=1)
  o_ref[...] = m_ref[...][:, :C] + agg + b2_ref[...]


def _final(m, a0, a1, b2r):
  blk = 1000
  grid = (N // blk,)
  hc = CP // 2
  return pl.pallas_call(
      _final_body,
      grid=grid,
      in_specs=[
          pl.BlockSpec((blk, CP), lambda i: (i, 0)),
          pl.BlockSpec((blk, hc), lambda i: (i, 0)),
          pl.BlockSpec((blk, hc), lambda i: (i, 0)),
          pl.BlockSpec((1, C), lambda i: (0, 0)),
      ],
      out_specs=pl.BlockSpec((blk, C), lambda i: (i, 0)),
      out_shape=jax.ShapeDtypeStruct((N, C), jnp.float32),
  )(m, a0, a1, b2r)


def kernel(x, edge_index, W1, b1, W2, b2):
  src = edge_index[0].astype(jnp.int32)
  dst = edge_index[1].astype(jnp.int32)
  pad = E_PAD - E
  dump = N + jnp.arange(pad, dtype=jnp.int32) % (N_ACC - N)
  src_p = jnp.concatenate([src, jnp.zeros((pad,), jnp.int32)]).reshape(-1, CH)
  dst_p = jnp.concatenate([dst, dump]).reshape(-1, CH)

  w2p = jnp.pad(W2, ((0, 0), (0, CP - C)))
  b1r = b1.reshape(1, H)
  b2r = b2.reshape(1, C)

  agg_x = _seg_sum_kernel(D // 2, 5)(x, src_p, dst_p)
  m = _mlp(x, agg_x[0], agg_x[1], W1, b1r, w2p)
  agg_m = _seg_sum_kernel(CP // 2, 10)(m, src_p, dst_p)
  return _final(m, agg_m[0], agg_m[1], b2r)


# speedup vs baseline: 1.0460x; 1.0460x over previous
import functools
import jax
import jax.numpy as jnp
from jax import lax
from jax.experimental import pallas as pl
from jax.experimental.pallas import tpu as pltpu
from jax.experimental.pallas import tpu_sc as plsc

N = 10000
E = 320000
D = 128
H = 128
C = 40
CP = 64

NC, NS = 2, 16
CH = 128
NCHT = 160
QC = 32
E_PAD = NS * NCHT * CH
N_ACC = 10112
ROWS_PT = N_ACC // NS
TROWS_PT = N // NS


@functools.lru_cache(maxsize=None)
def _seg_sum_kernel(FH, nbuf):
  assert QC % nbuf == 0 and NCHT % QC == 0
  mesh = plsc.VectorSubcoreMesh(
      core_axis_name="c", subcore_axis_name="s", num_cores=NC, num_subcores=NS)

  @functools.partial(
      pl.kernel,
      out_type=jax.ShapeDtypeStruct((NC, N_ACC, FH), jnp.float32),
      mesh=mesh,
      scratch_types=(
          [
              pltpu.VMEM((QC, CH), jnp.int32),
              pltpu.VMEM((QC, CH), jnp.int32),
          ]
          + [pltpu.VMEM((CH, FH), jnp.float32) for _ in range(nbuf)]
          + [
              pltpu.VMEM_SHARED((N, FH), jnp.float32),
              pltpu.VMEM_SHARED((N_ACC, FH), jnp.float32),
          ]
          + [pltpu.SemaphoreType.DMA for _ in range(nbuf)]
      ),
      compiler_params=pltpu.CompilerParams(use_tc_tiling_on_sc=False),
  )
  def seg_sum(h2_hbm, src_hbm, dst_hbm, out_hbm, src_v, dst_v, *rest):
    bufs = rest[:nbuf]
    table = rest[nbuf]
    acc = rest[nbuf + 1]
    sems = rest[nbuf + 2:]
    c = lax.axis_index("c")
    s = lax.axis_index("s")

    zv = jnp.zeros((16,), jnp.float32)

    with jax.named_scope("stage"):
      def zrow(i, carry):
        for k in range(FH // 16):
          bufs[0][i, pl.ds(k * 16, 16)] = zv
        return carry

      lax.fori_loop(0, CH, zrow, 0)

      base = s * ROWS_PT
      for r in range(ROWS_PT // CH):
        pltpu.sync_copy(bufs[0], acc.at[pl.ds(base + r * CH, CH)])
      rem = ROWS_PT % CH
      if rem:
        pltpu.sync_copy(bufs[0].at[pl.ds(0, rem)],
                        acc.at[pl.ds(base + (ROWS_PT // CH) * CH, rem)])

      tbase = s * TROWS_PT
      pltpu.sync_copy(h2_hbm.at[pl.ds(tbase, TROWS_PT), pl.ds(c * FH, FH)],
                      table.at[pl.ds(tbase, TROWS_PT)])

      plsc.subcore_barrier()

    def wait_gather(k):
      pltpu.make_async_copy(table.at[pl.ds(0, CH)], bufs[k], sems[k]).wait()

    with jax.named_scope("edges"):
      for g in range(NCHT // QC):
        chunk0 = s * NCHT + g * QC
        pltpu.sync_copy(src_hbm.at[pl.ds(chunk0, QC)], src_v)
        pltpu.sync_copy(dst_hbm.at[pl.ds(chunk0, QC)], dst_v)

        for k in range(nbuf):
          pltpu.async_copy(table.at[src_v.at[k]], bufs[k], sems[k])

        def body(q, carry):
          j = q * nbuf
          for k in range(nbuf):
            wait_gather(k)
            pltpu.sync_copy(bufs[k], acc.at[dst_v.at[j + k]], add=True)
            pltpu.async_copy(table.at[src_v.at[j + k + nbuf]], bufs[k], sems[k])
          return carry

        lax.fori_loop(0, QC // nbuf - 1, body, 0)

        j = QC - nbuf
        for k in range(nbuf):
          wait_gather(k)
          pltpu.sync_copy(bufs[k], acc.at[dst_v.at[j + k]], add=True)

      plsc.subcore_barrier()

    with jax.named_scope("copyout"):
      pltpu.sync_copy(acc.at[pl.ds(base, ROWS_PT)],
                      out_hbm.at[c, pl.ds(base, ROWS_PT)])

  return seg_sum


def _mlp_body(x_ref, a0_ref, a1_ref, w1_ref, b1_ref, w2_ref, m_ref):
  agg = jnp.concatenate([a0_ref[...], a1_ref[...]], axis=1)
  t = x_ref[...] + agg
  h = jnp.dot(t, w1_ref[...], preferred_element_type=jnp.float32) + b1_ref[...]
  h = jnp.maximum(h, 0.0)
  m_ref[...] = jnp.dot(h, w2_ref[...], preferred_element_type=jnp.float32)


def _mlp(x, a0, a1, w1, b1, w2p):
  blk = 1000
  grid = (N // blk,)
  hd = D // 2
  return pl.pallas_call(
      _mlp_body,
      grid=grid,
      in_specs=[
          pl.BlockSpec((blk, D), lambda i: (i, 0)),
          pl.BlockSpec((blk, hd), lambda i: (i, 0)),
          pl.BlockSpec((blk, hd), lambda i: (i, 0)),
          pl.BlockSpec((D, H), lambda i: (0, 0)),
          pl.BlockSpec((1, H), lambda i: (0, 0)),
          pl.BlockSpec((H, CP), lambda i: (0, 0)),
      ],
      out_specs=pl.BlockSpec((blk, CP), lambda i: (i, 0)),
      out_shape=jax.ShapeDtypeStruct((N, CP), jnp.float32),
  )(x, a0, a1, w1, b1, w2p)


def _final_body(m_ref, a0_ref, a1_ref, b2_ref, o_ref):
  agg = jnp.concatenate([a0_ref[...], a1_ref[...][:, :C - CP // 2]], axis=1)
  o_ref[...] = m_ref[...][:, :C] + agg + b2_ref[...]


def _final(m, a0, a1, b2r):
  blk = 1000
  grid = (N // blk,)
  hc = CP // 2
  return pl.pallas_call(
      _final_body,
      grid=grid,
      in_specs=[
          pl.BlockSpec((blk, CP), lambda i: (i, 0)),
          pl.BlockSpec((blk, hc), lambda i: (i, 0)),
          pl.BlockSpec((blk, hc), lambda i: (i, 0)),
          pl.BlockSpec((1, C), lambda i: (0, 0)),
      ],
      out_specs=pl.BlockSpec((blk, C), lambda i: (i, 0)),
      out_shape=jax.ShapeDtypeStruct((N, C), jnp.float32),
  )(m, a0, a1, b2r)


def kernel(x, edge_index, W1, b1, W2, b2):
  src = edge_index[0].astype(jnp.int32)
  dst = edge_index[1].astype(jnp.int32)
  pad = E_PAD - E
  dump = N + jnp.arange(pad, dtype=jnp.int32) % (N_ACC - N)
  src_p = jnp.concatenate([src, jnp.zeros((pad,), jnp.int32)]).reshape(-1, CH)
  dst_p = jnp.concatenate([dst, dump]).reshape(-1, CH)

  w2p = jnp.pad(W2, ((0, 0), (0, CP - C)))
  b1r = b1.reshape(1, H)
  b2r = b2.reshape(1, C)

  agg_x = _seg_sum_kernel(D // 2, 4)(x, src_p, dst_p)
  m = _mlp(x, agg_x[0], agg_x[1], W1, b1r, w2p)
  agg_m = _seg_sum_kernel(CP // 2, 8)(m, src_p, dst_p)
  return _final(m, agg_m[0], agg_m[1], b2r)


# speedup vs baseline: 1.1097x; 1.0609x over previous
import functools
import jax
import jax.numpy as jnp
from jax import lax
from jax.experimental import pallas as pl
from jax.experimental.pallas import tpu as pltpu
from jax.experimental.pallas import tpu_sc as plsc

N = 10000
E = 320000
D = 128
H = 128
C = 40
CP = 64

NC, NS = 2, 16
CH = 128
NCHT = 160
QC = 40
E_PAD = NS * NCHT * CH
N_ACC = 10112
ROWS_PT = N_ACC // NS
TROWS_PT = N // NS


@functools.lru_cache(maxsize=None)
def _seg_sum_kernel(FH, nbuf):
  assert QC % nbuf == 0 and NCHT % QC == 0
  mesh = plsc.VectorSubcoreMesh(
      core_axis_name="c", subcore_axis_name="s", num_cores=NC, num_subcores=NS)

  @functools.partial(
      pl.kernel,
      out_type=jax.ShapeDtypeStruct((NC, N_ACC, FH), jnp.float32),
      mesh=mesh,
      scratch_types=(
          [
              pltpu.VMEM((QC, CH), jnp.int32),
              pltpu.VMEM((QC, CH), jnp.int32),
          ]
          + [pltpu.VMEM((CH, FH), jnp.float32) for _ in range(nbuf)]
          + [
              pltpu.VMEM_SHARED((N, FH), jnp.float32),
              pltpu.VMEM_SHARED((N_ACC, FH), jnp.float32),
          ]
          + [pltpu.SemaphoreType.DMA for _ in range(nbuf)]
      ),
      compiler_params=pltpu.CompilerParams(use_tc_tiling_on_sc=False),
  )
  def seg_sum(h2_hbm, src_hbm, dst_hbm, out_hbm, src_v, dst_v, *rest):
    bufs = rest[:nbuf]
    table = rest[nbuf]
    acc = rest[nbuf + 1]
    sems = rest[nbuf + 2:]
    c = lax.axis_index("c")
    s = lax.axis_index("s")

    zv = jnp.zeros((16,), jnp.float32)

    with jax.named_scope("stage"):
      def zrow(i, carry):
        for k in range(FH // 16):
          bufs[0][i, pl.ds(k * 16, 16)] = zv
        return carry

      lax.fori_loop(0, CH, zrow, 0)

      base = s * ROWS_PT
      for r in range(ROWS_PT // CH):
        pltpu.sync_copy(bufs[0], acc.at[pl.ds(base + r * CH, CH)])
      rem = ROWS_PT % CH
      if rem:
        pltpu.sync_copy(bufs[0].at[pl.ds(0, rem)],
                        acc.at[pl.ds(base + (ROWS_PT // CH) * CH, rem)])

      tbase = s * TROWS_PT
      pltpu.sync_copy(h2_hbm.at[pl.ds(tbase, TROWS_PT), pl.ds(c * FH, FH)],
                      table.at[pl.ds(tbase, TROWS_PT)])

      plsc.subcore_barrier()

    def wait_gather(k):
      pltpu.make_async_copy(table.at[pl.ds(0, CH)], bufs[k], sems[k]).wait()

    with jax.named_scope("edges"):
      for g in range(NCHT // QC):
        chunk0 = s * NCHT + g * QC
        pltpu.sync_copy(src_hbm.at[pl.ds(chunk0, QC)], src_v)
        pltpu.sync_copy(dst_hbm.at[pl.ds(chunk0, QC)], dst_v)

        for k in range(nbuf):
          pltpu.async_copy(table.at[src_v.at[k]], bufs[k], sems[k])

        def body(q, carry):
          j = q * nbuf
          for k in range(nbuf):
            wait_gather(k)
            pltpu.sync_copy(bufs[k], acc.at[dst_v.at[j + k]], add=True)
            pltpu.async_copy(table.at[src_v.at[j + k + nbuf]], bufs[k], sems[k])
          return carry

        lax.fori_loop(0, QC // nbuf - 1, body, 0)

        j = QC - nbuf
        for k in range(nbuf):
          wait_gather(k)
          pltpu.sync_copy(bufs[k], acc.at[dst_v.at[j + k]], add=True)

      plsc.subcore_barrier()

    with jax.named_scope("copyout"):
      pltpu.sync_copy(acc.at[pl.ds(base, ROWS_PT)],
                      out_hbm.at[c, pl.ds(base, ROWS_PT)])

  return seg_sum


def _mlp_body(x_ref, a0_ref, a1_ref, w1_ref, b1_ref, w2_ref, m_ref):
  agg = jnp.concatenate([a0_ref[0], a1_ref[0]], axis=1)
  t = x_ref[...] + agg
  h = jnp.dot(t, w1_ref[...], preferred_element_type=jnp.float32) + b1_ref[...]
  h = jnp.maximum(h, 0.0)
  m_ref[...] = jnp.dot(h, w2_ref[...], preferred_element_type=jnp.float32)


def _mlp(x, agg, w1, b1, w2p):
  blk = 1000
  grid = (N // blk,)
  hd = D // 2
  return pl.pallas_call(
      _mlp_body,
      grid=grid,
      in_specs=[
          pl.BlockSpec((blk, D), lambda i: (i, 0)),
          pl.BlockSpec((1, blk, hd), lambda i: (0, i, 0)),
          pl.BlockSpec((1, blk, hd), lambda i: (1, i, 0)),
          pl.BlockSpec((D, H), lambda i: (0, 0)),
          pl.BlockSpec((1, H), lambda i: (0, 0)),
          pl.BlockSpec((H, CP), lambda i: (0, 0)),
      ],
      out_specs=pl.BlockSpec((blk, CP), lambda i: (i, 0)),
      out_shape=jax.ShapeDtypeStruct((N, CP), jnp.float32),
  )(x, agg, agg, w1, b1, w2p)


def _final_body(m_ref, a0_ref, a1_ref, b2_ref, o_ref):
  agg = jnp.concatenate([a0_ref[0], a1_ref[0][:, :C - CP // 2]], axis=1)
  o_ref[...] = m_ref[...][:, :C] + agg + b2_ref[...]


def _final(m, agg, b2r):
  blk = 1000
  grid = (N // blk,)
  hc = CP // 2
  return pl.pallas_call(
      _final_body,
      grid=grid,
      in_specs=[
          pl.BlockSpec((blk, CP), lambda i: (i, 0)),
          pl.BlockSpec((1, blk, hc), lambda i: (0, i, 0)),
          pl.BlockSpec((1, blk, hc), lambda i: (1, i, 0)),
          pl.BlockSpec((1, C), lambda i: (0, 0)),
      ],
      out_specs=pl.BlockSpec((blk, C), lambda i: (i, 0)),
      out_shape=jax.ShapeDtypeStruct((N, C), jnp.float32),
  )(m, agg, agg, b2r)


def kernel(x, edge_index, W1, b1, W2, b2):
  src = edge_index[0].astype(jnp.int32)
  dst = edge_index[1].astype(jnp.int32)
  pad = E_PAD - E
  dump = N + jnp.arange(pad, dtype=jnp.int32) % (N_ACC - N)
  src_p = jnp.concatenate([src, jnp.zeros((pad,), jnp.int32)]).reshape(-1, CH)
  dst_p = jnp.concatenate([dst, dump]).reshape(-1, CH)

  w2p = jnp.pad(W2, ((0, 0), (0, CP - C)))
  b1r = b1.reshape(1, H)
  b2r = b2.reshape(1, C)

  agg_x = _seg_sum_kernel(D // 2, 4)(x, src_p, dst_p)
  m = _mlp(x, agg_x, W1, b1r, w2p)
  agg_m = _seg_sum_kernel(CP // 2, 8)(m, src_p, dst_p)
  return _final(m, agg_m, b2r)


# speedup vs baseline: 1.1278x; 1.0163x over previous
import functools
import jax
import jax.numpy as jnp
from jax import lax
from jax.experimental import pallas as pl
from jax.experimental.pallas import tpu as pltpu
from jax.experimental.pallas import tpu_sc as plsc

N = 10000
E = 320000
D = 128
H = 128
C = 40
CP = 64

NC, NS = 2, 16
CH = 128
NCHT = 160
E_PAD = NS * NCHT * CH
N_ACC = 10112
ROWS_PT = N_ACC // NS
TROWS_PT = N // NS


@functools.lru_cache(maxsize=None)
def _seg_sum_kernel(FH, nbuf, QC):
  assert QC % nbuf == 0 and NCHT % QC == 0
  mesh = plsc.VectorSubcoreMesh(
      core_axis_name="c", subcore_axis_name="s", num_cores=NC, num_subcores=NS)

  @functools.partial(
      pl.kernel,
      out_type=jax.ShapeDtypeStruct((NC, N_ACC, FH), jnp.float32),
      mesh=mesh,
      scratch_types=(
          [
              pltpu.VMEM((QC, CH), jnp.int32),
              pltpu.VMEM((QC, CH), jnp.int32),
          ]
          + [pltpu.VMEM((CH, FH), jnp.float32) for _ in range(nbuf)]
          + [
              pltpu.VMEM_SHARED((N, FH), jnp.float32),
              pltpu.VMEM_SHARED((N_ACC, FH), jnp.float32),
          ]
          + [pltpu.SemaphoreType.DMA for _ in range(nbuf)]
      ),
      compiler_params=pltpu.CompilerParams(use_tc_tiling_on_sc=False),
  )
  def seg_sum(h2_hbm, src_hbm, dst_hbm, out_hbm, src_v, dst_v, *rest):
    bufs = rest[:nbuf]
    table = rest[nbuf]
    acc = rest[nbuf + 1]
    sems = rest[nbuf + 2:]
    c = lax.axis_index("c")
    s = lax.axis_index("s")

    zv = jnp.zeros((16,), jnp.float32)

    with jax.named_scope("stage"):
      def zrow(i, carry):
        for k in range(FH // 16):
          bufs[0][i, pl.ds(k * 16, 16)] = zv
        return carry

      lax.fori_loop(0, CH, zrow, 0)

      base = s * ROWS_PT
      for r in range(ROWS_PT // CH):
        pltpu.sync_copy(bufs[0], acc.at[pl.ds(base + r * CH, CH)])
      rem = ROWS_PT % CH
      if rem:
        pltpu.sync_copy(bufs[0].at[pl.ds(0, rem)],
                        acc.at[pl.ds(base + (ROWS_PT // CH) * CH, rem)])

      tbase = s * TROWS_PT
      pltpu.sync_copy(h2_hbm.at[pl.ds(tbase, TROWS_PT), pl.ds(c * FH, FH)],
                      table.at[pl.ds(tbase, TROWS_PT)])

      plsc.subcore_barrier()

    def wait_gather(k):
      pltpu.make_async_copy(table.at[pl.ds(0, CH)], bufs[k], sems[k]).wait()

    with jax.named_scope("edges"):
      for g in range(NCHT // QC):
        chunk0 = s * NCHT + g * QC
        pltpu.sync_copy(src_hbm.at[pl.ds(chunk0, QC)], src_v)
        pltpu.sync_copy(dst_hbm.at[pl.ds(chunk0, QC)], dst_v)

        for k in range(nbuf):
          pltpu.async_copy(table.at[src_v.at[k]], bufs[k], sems[k])

        def body(q, carry):
          j = q * nbuf
          for k in range(nbuf):
            wait_gather(k)
            pltpu.sync_copy(bufs[k], acc.at[dst_v.at[j + k]], add=True)
            pltpu.async_copy(table.at[src_v.at[j + k + nbuf]], bufs[k], sems[k])
          return carry

        lax.fori_loop(0, QC // nbuf - 1, body, 0)

        j = QC - nbuf
        for k in range(nbuf):
          wait_gather(k)
          pltpu.sync_copy(bufs[k], acc.at[dst_v.at[j + k]], add=True)

      plsc.subcore_barrier()

    with jax.named_scope("copyout"):
      pltpu.sync_copy(acc.at[pl.ds(base, ROWS_PT)],
                      out_hbm.at[c, pl.ds(base, ROWS_PT)])

  return seg_sum


def _mlp_body(x_ref, a0_ref, a1_ref, w1_ref, b1_ref, w2_ref, m_ref):
  agg = jnp.concatenate([a0_ref[0], a1_ref[0]], axis=1)
  t = x_ref[...] + agg
  h = jnp.dot(t, w1_ref[...], preferred_element_type=jnp.float32) + b1_ref[...]
  h = jnp.maximum(h, 0.0)
  m_ref[...] = jnp.dot(h, w2_ref[...], preferred_element_type=jnp.float32)


def _mlp(x, agg, w1, b1, w2p):
  blk = 1000
  grid = (N // blk,)
  hd = D // 2
  return pl.pallas_call(
      _mlp_body,
      grid=grid,
      in_specs=[
          pl.BlockSpec((blk, D), lambda i: (i, 0)),
          pl.BlockSpec((1, blk, hd), lambda i: (0, i, 0)),
          pl.BlockSpec((1, blk, hd), lambda i: (1, i, 0)),
          pl.BlockSpec((D, H), lambda i: (0, 0)),
          pl.BlockSpec((1, H), lambda i: (0, 0)),
          pl.BlockSpec((H, CP), lambda i: (0, 0)),
      ],
      out_specs=pl.BlockSpec((blk, CP), lambda i: (i, 0)),
      out_shape=jax.ShapeDtypeStruct((N, CP), jnp.float32),
  )(x, agg, agg, w1, b1, w2p)


def _final_body(m_ref, a0_ref, a1_ref, b2_ref, o_ref):
  agg = jnp.concatenate([a0_ref[0], a1_ref[0][:, :C - CP // 2]], axis=1)
  o_ref[...] = m_ref[...][:, :C] + agg + b2_ref[...]


def _final(m, agg, b2r):
  blk = 1000
  grid = (N // blk,)
  hc = CP // 2
  return pl.pallas_call(
      _final_body,
      grid=grid,
      in_specs=[
          pl.BlockSpec((blk, CP), lambda i: (i, 0)),
          pl.BlockSpec((1, blk, hc), lambda i: (0, i, 0)),
          pl.BlockSpec((1, blk, hc), lambda i: (1, i, 0)),
          pl.BlockSpec((1, C), lambda i: (0, 0)),
      ],
      out_specs=pl.BlockSpec((blk, C), lambda i: (i, 0)),
      out_shape=jax.ShapeDtypeStruct((N, C), jnp.float32),
  )(m, agg, agg, b2r)


def kernel(x, edge_index, W1, b1, W2, b2):
  src = edge_index[0].astype(jnp.int32)
  dst = edge_index[1].astype(jnp.int32)
  pad = E_PAD - E
  dump = N + jnp.arange(pad, dtype=jnp.int32) % (N_ACC - N)
  src_p = jnp.concatenate([src, jnp.zeros((pad,), jnp.int32)]).reshape(-1, CH)
  dst_p = jnp.concatenate([dst, dump]).reshape(-1, CH)

  w2p = jnp.pad(W2, ((0, 0), (0, CP - C)))
  b1r = b1.reshape(1, H)
  b2r = b2.reshape(1, C)

  agg_x = _seg_sum_kernel(D // 2, 4, 40)(x, src_p, dst_p)
  m = _mlp(x, agg_x, W1, b1r, w2p)
  agg_m = _seg_sum_kernel(CP // 2, 8, 80)(m, src_p, dst_p)
  return _final(m, agg_m, b2r)


# speedup vs baseline: 1.1601x; 1.0287x over previous
import functools
import jax
import jax.numpy as jnp
from jax import lax
from jax.experimental import pallas as pl
from jax.experimental.pallas import tpu as pltpu
from jax.experimental.pallas import tpu_sc as plsc

N = 10000
E = 320000
D = 128
H = 128
C = 40
CP = 64

NC, NS = 2, 16
CH = 128
NCHT = 160
E_PAD = NS * NCHT * CH
N_ACC = 10112
ROWS_PT = N_ACC // NS
TROWS_PT = N // NS


@functools.lru_cache(maxsize=None)
def _seg_sum_kernel(FH, nbuf, QC):
  assert QC % nbuf == 0 and NCHT % QC == 0
  mesh = plsc.VectorSubcoreMesh(
      core_axis_name="c", subcore_axis_name="s", num_cores=NC, num_subcores=NS)

  @functools.partial(
      pl.kernel,
      out_type=jax.ShapeDtypeStruct((NC, N_ACC, FH), jnp.float32),
      mesh=mesh,
      scratch_types=(
          [
              pltpu.VMEM((QC, CH), jnp.int32),
              pltpu.VMEM((QC, CH), jnp.int32),
          ]
          + [pltpu.VMEM((CH, FH), jnp.float32) for _ in range(nbuf)]
          + [
              pltpu.VMEM_SHARED((N, FH), jnp.float32),
              pltpu.VMEM_SHARED((N_ACC, FH), jnp.float32),
          ]
          + [pltpu.SemaphoreType.DMA for _ in range(nbuf)]
      ),
      compiler_params=pltpu.CompilerParams(use_tc_tiling_on_sc=False),
  )
  def seg_sum(h2_hbm, src_hbm, dst_hbm, out_hbm, src_v, dst_v, *rest):
    bufs = rest[:nbuf]
    table = rest[nbuf]
    acc = rest[nbuf + 1]
    sems = rest[nbuf + 2:]
    c = lax.axis_index("c")
    s = lax.axis_index("s")

    zv = jnp.zeros((16,), jnp.float32)

    with jax.named_scope("stage"):
      def zrow(i, carry):
        for k in range(FH // 16):
          bufs[0][i, pl.ds(k * 16, 16)] = zv
        return carry

      lax.fori_loop(0, CH, zrow, 0)

      base = s * ROWS_PT
      for r in range(ROWS_PT // CH):
        pltpu.sync_copy(bufs[0], acc.at[pl.ds(base + r * CH, CH)])
      rem = ROWS_PT % CH
      if rem:
        pltpu.sync_copy(bufs[0].at[pl.ds(0, rem)],
                        acc.at[pl.ds(base + (ROWS_PT // CH) * CH, rem)])

      tbase = s * TROWS_PT
      pltpu.sync_copy(h2_hbm.at[pl.ds(tbase, TROWS_PT), pl.ds(c * FH, FH)],
                      table.at[pl.ds(tbase, TROWS_PT)])

      plsc.subcore_barrier()

    def wait_gather(k):
      pltpu.make_async_copy(table.at[pl.ds(0, CH)], bufs[k], sems[k]).wait()

    with jax.named_scope("edges"):
      for g in range(NCHT // QC):
        chunk0 = s * NCHT + g * QC
        pltpu.sync_copy(src_hbm.at[pl.ds(chunk0, QC)], src_v)
        pltpu.sync_copy(dst_hbm.at[pl.ds(chunk0, QC)], dst_v)

        for k in range(nbuf):
          pltpu.async_copy(table.at[src_v.at[k]], bufs[k], sems[k])

        def body(q, carry):
          j = q * nbuf
          for k in range(nbuf):
            wait_gather(k)
            pltpu.sync_copy(bufs[k], acc.at[dst_v.at[j + k]], add=True)
            pltpu.async_copy(table.at[src_v.at[j + k + nbuf]], bufs[k], sems[k])
          return carry

        lax.fori_loop(0, QC // nbuf - 1, body, 0)

        j = QC - nbuf
        for k in range(nbuf):
          wait_gather(k)
          pltpu.sync_copy(bufs[k], acc.at[dst_v.at[j + k]], add=True)

      plsc.subcore_barrier()

    with jax.named_scope("copyout"):
      pltpu.sync_copy(acc.at[pl.ds(base, ROWS_PT)],
                      out_hbm.at[c, pl.ds(base, ROWS_PT)])

  return seg_sum


def _mlp_body(x_ref, a0_ref, a1_ref, w1_ref, b1_ref, w2_ref, m_ref):
  agg = jnp.concatenate([a0_ref[0], a1_ref[0]], axis=1)
  t = x_ref[...] + agg
  h = jnp.dot(t, w1_ref[...], preferred_element_type=jnp.float32) + b1_ref[...]
  h = jnp.maximum(h, 0.0)
  m_ref[...] = jnp.dot(h, w2_ref[...], preferred_element_type=jnp.float32)


def _mlp(x, agg, w1, b1, w2p):
  blk = 1000
  grid = (N // blk,)
  hd = D // 2
  return pl.pallas_call(
      _mlp_body,
      grid=grid,
      in_specs=[
          pl.BlockSpec((blk, D), lambda i: (i, 0)),
          pl.BlockSpec((1, blk, hd), lambda i: (0, i, 0)),
          pl.BlockSpec((1, blk, hd), lambda i: (1, i, 0)),
          pl.BlockSpec((D, H), lambda i: (0, 0)),
          pl.BlockSpec((1, H), lambda i: (0, 0)),
          pl.BlockSpec((H, CP), lambda i: (0, 0)),
      ],
      out_specs=pl.BlockSpec((blk, CP), lambda i: (i, 0)),
      out_shape=jax.ShapeDtypeStruct((N, CP), jnp.float32),
  )(x, agg, agg, w1, b1, w2p)


def _final_body(m_ref, a0_ref, a1_ref, b2_ref, o_ref):
  agg = jnp.concatenate([a0_ref[0], a1_ref[0][:, :C - CP // 2]], axis=1)
  o_ref[...] = m_ref[...][:, :C] + agg + b2_ref[...]


def _final(m, agg, b2r):
  blk = 1000
  grid = (N // blk,)
  hc = CP // 2
  return pl.pallas_call(
      _final_body,
      grid=grid,
      in_specs=[
          pl.BlockSpec((blk, CP), lambda i: (i, 0)),
          pl.BlockSpec((1, blk, hc), lambda i: (0, i, 0)),
          pl.BlockSpec((1, blk, hc), lambda i: (1, i, 0)),
          pl.BlockSpec((1, C), lambda i: (0, 0)),
      ],
      out_specs=pl.BlockSpec((blk, C), lambda i: (i, 0)),
      out_shape=jax.ShapeDtypeStruct((N, C), jnp.float32),
  )(m, agg, agg, b2r)


def kernel(x, edge_index, W1, b1, W2, b2):
  src = edge_index[0].astype(jnp.int32)
  dst = edge_index[1].astype(jnp.int32)
  pad = E_PAD - E
  dump = N + jnp.arange(pad, dtype=jnp.int32) % (N_ACC - N)
  src_p = jnp.concatenate([src, jnp.zeros((pad,), jnp.int32)]).reshape(-1, CH)
  dst_p = jnp.concatenate([dst, dump]).reshape(-1, CH)

  w2p = jnp.pad(W2, ((0, 0), (0, CP - C)))
  b1r = b1.reshape(1, H)
  b2r = b2.reshape(1, C)

  agg_x = _seg_sum_kernel(D // 2, 2, 80)(x, src_p, dst_p)
  m = _mlp(x, agg_x, W1, b1r, w2p)
  agg_m = _seg_sum_kernel(CP // 2, 8, 80)(m, src_p, dst_p)
  return _final(m, agg_m, b2r)
